# baseline (speedup 1.0000x reference)
import jax
import jax.numpy as jnp
from jax import lax
from jax.experimental import pallas as pl
from jax.experimental.pallas import tpu as pltpu
import jax.experimental.pallas.tpu_sc as plsc

N = 100000
E = 3200000
D_IN = 128
D_HID = 16
D_OUT = 40

NC = 2
NS = 16
LANE = 128
EPAD = 3276800
ROWS = EPAD // LANE
ROWS_W = ROWS // (NC * NS)
NPAD = 102400
NSLICE = NPAD // NS
TRASH = NPAD - N

CH = 5
NCHUNK = ROWS_W // CH
DCH = 16
ZROWS = 100
RV = 400

PR = N * D_HID // 128
PRA = NPAD * D_HID // 128
BLKR = 256


def _deg_body(dst_hbm, out_hbm, accum, zbuf, ones, dstbuf, deg_v, rep_v, sem):
  c = lax.axis_index("c")
  s = lax.axis_index("s")

  @pl.loop(0, NSLICE // 16)
  def _zfill(i):
    zbuf[pl.ds(i * 16, 16)] = jnp.zeros((16,), jnp.float32)

  @pl.loop(0, LANE // 16)
  def _ofill(i):
    ones[pl.ds(i * 16, 16)] = jnp.ones((16,), jnp.float32)

  pltpu.sync_copy(zbuf, accum.at[pl.ds(s * NSLICE, NSLICE)])
  plsc.subcore_barrier()

  base = (c * NS + s) * ROWS_W

  @pl.loop(0, ROWS_W // DCH)
  def _chunk(ch):
    r0 = base + ch * DCH
    pltpu.sync_copy(dst_hbm.at[pl.ds(r0, DCH)], dstbuf)
    descs = []
    for j in range(DCH):
      descs.append(pltpu.async_copy(ones, accum.at[dstbuf.at[j]], sem, add=True))
    for d in descs:
      d.wait()

  plsc.subcore_barrier()

  @pl.loop(0, NSLICE // RV)
  def _rep(it):
    pltpu.sync_copy(accum.at[pl.ds(s * NSLICE + it * RV, RV)], deg_v)

    @pl.loop(0, RV // 16)
    def _grp(g):
      v = deg_v[pl.ds(g * 16, 16)]
      for j in range(16):
        rep_v[pl.ds((g * 16 + j) * 16, 16)] = v.at[
            jnp.full((16,), j, jnp.int32)].get(mode="promise_in_bounds")

    pltpu.sync_copy(
        rep_v,
        out_hbm.at[pl.ds((c * NPAD + s * NSLICE + it * RV) * D_HID,
                         RV * D_HID)])


def _agg_body(table_hbm, src_hbm, dst_hbm, out_hbm, accum, zbuf, srcA, dstA,
              rowA, srcB, dstB, rowB, isemA, isemB, gsem, ssem):
  c = lax.axis_index("c")
  s = lax.axis_index("s")

  @pl.loop(0, ZROWS)
  def _zfill(i):
    zbuf[i, :] = jnp.zeros((D_HID,), jnp.float32)

  @pl.loop(0, NSLICE // ZROWS)
  def _zinit(j):
    pltpu.sync_copy(zbuf, accum.at[pl.ds(s * NSLICE + j * ZROWS, ZROWS), :])

  plsc.subcore_barrier()

  base = (c * NS + s) * ROWS_W

  def _stage(ch, sbuf, dbuf, sem):
    r0 = base + ch * CH
    pltpu.async_copy(src_hbm.at[pl.ds(r0, CH)], sbuf, sem)
    pltpu.async_copy(dst_hbm.at[pl.ds(r0, CH)], dbuf, sem)

  def _drain_idx(sbuf, dbuf, sem):
    pltpu.make_async_copy(src_hbm.at[pl.ds(0, CH)], sbuf, sem).wait()
    pltpu.make_async_copy(dst_hbm.at[pl.ds(0, CH)], dbuf, sem).wait()

  _stage(0, srcA, dstA, isemA)
  _stage(1, srcB, dstB, isemB)

  @pl.loop(0, NCHUNK // 2)
  def _pair(i):
    _drain_idx(srcA, dstA, isemA)
    gdA = [pltpu.async_copy(table_hbm.at[srcA.at[j]], rowA.at[j], gsem)
           for j in range(CH)]
    sdA = []
    for j in range(CH):
      gdA[j].wait()
      sdA.append(pltpu.async_copy(rowA.at[j], accum.at[dstA.at[j]], ssem,
                                  add=True))
    _drain_idx(srcB, dstB, isemB)
    gdB = [pltpu.async_copy(table_hbm.at[srcB.at[j]], rowB.at[j], gsem)
           for j in range(CH)]
    for d in sdA:
      d.wait()
    _stage(jnp.minimum(2 * i + 2, NCHUNK - 1), srcA, dstA, isemA)
    sdB = []
    for j in range(CH):
      gdB[j].wait()
      sdB.append(pltpu.async_copy(rowB.at[j], accum.at[dstB.at[j]], ssem,
                                  add=True))
    for d in sdB:
      d.wait()
    _stage(jnp.minimum(2 * i + 3, NCHUNK - 1), srcB, dstB, isemB)

  _drain_idx(srcA, dstA, isemA)
  _drain_idx(srcB, dstB, isemB)
  plsc.subcore_barrier()
  pltpu.sync_copy(accum.at[pl.ds(s * NSLICE, NSLICE), :],
                  out_hbm.at[pl.ds(c * NPAD + s * NSLICE, NSLICE), :])


def _ka_body(x_ref, w1_ref, h_ref):
  for j in range(8):
    h_ref[:, pl.ds(D_HID * j, D_HID)] = jnp.dot(
        x_ref[:, j, :], w1_ref[...], preferred_element_type=jnp.float32)


def _kb_body(h_ref, degr_ref, g1_ref, dis_ref):
  degsum = degr_ref[0] + degr_ref[1] + 1.0
  dis = lax.rsqrt(degsum)
  g1_ref[...] = h_ref[...] * dis
  dis_ref[...] = dis


def _k4_body(a_ref, g1_ref, dis_ref, b1p_ref, o_ref):
  dis = dis_ref[...]
  t = dis * (a_ref[0] + a_ref[1] + g1_ref[...]) + b1p_ref[...]
  o_ref[...] = dis * jnp.maximum(t, 0.0)


def _k6_body(a_ref, g1r_ref, dis_ref, w2_ref, b2_ref, o_ref):
  m = dis_ref[...] * (a_ref[0] + a_ref[1] + g1r_ref[...])
  cols = [
      jnp.dot(m[:, D_HID * j:D_HID * (j + 1)], w2_ref[...],
              preferred_element_type=jnp.float32)
      for j in range(8)
  ]
  stacked = jnp.stack(cols, axis=1)
  o_ref[...] = stacked.reshape(8 * BLKR, D_OUT) + b2_ref[...]


def _sc_mesh():
  return plsc.VectorSubcoreMesh(core_axis_name="c", subcore_axis_name="s",
                                num_cores=NC, num_subcores=NS)


_SC_PARAMS = pltpu.CompilerParams(use_tc_tiling_on_sc=False)


def _deg_call(dst_p):
  fn = pl.kernel(
      _deg_body,
      out_type=jax.ShapeDtypeStruct((NC * NPAD * D_HID,), jnp.float32),
      mesh=_sc_mesh(),
      compiler_params=_SC_PARAMS,
      scratch_types=[
          pltpu.VMEM_SHARED((NPAD,), jnp.float32),
          pltpu.VMEM((NSLICE,), jnp.float32),
          pltpu.VMEM((LANE,), jnp.float32),
          pltpu.VMEM((DCH, LANE), jnp.int32),
          pltpu.VMEM((RV,), jnp.float32),
          pltpu.VMEM((RV * D_HID,), jnp.float32),
          pltpu.SemaphoreType.DMA,
      ],
  )
  return fn(dst_p)


def _agg_call(table, src_p, dst_p):
  fn = pl.kernel(
      _agg_body,
      out_type=jax.ShapeDtypeStruct((NC * NPAD, D_HID), jnp.float32),
      mesh=_sc_mesh(),
      compiler_params=_SC_PARAMS,
      scratch_types=[
          pltpu.VMEM_SHARED((NPAD, D_HID), jnp.float32),
          pltpu.VMEM((ZROWS, D_HID), jnp.float32),
          pltpu.VMEM((CH, LANE), jnp.int32),
          pltpu.VMEM((CH, LANE), jnp.int32),
          pltpu.VMEM((CH, LANE, D_HID), jnp.float32),
          pltpu.VMEM((CH, LANE), jnp.int32),
          pltpu.VMEM((CH, LANE), jnp.int32),
          pltpu.VMEM((CH, LANE, D_HID), jnp.float32),
          pltpu.SemaphoreType.DMA,
          pltpu.SemaphoreType.DMA,
          pltpu.SemaphoreType.DMA,
          pltpu.SemaphoreType.DMA,
      ],
  )
  return fn(table, src_p, dst_p)


def kernel(x, edge_index, W1, b1, W2, b2):
  src = edge_index[0].astype(jnp.int32)
  dst = edge_index[1].astype(jnp.int32)
  padlen = EPAD - E
  pad_ids = jnp.arange(padlen, dtype=jnp.int32)
  src_p = jnp.concatenate([src, pad_ids % 4096]).reshape(ROWS, LANE)
  dst_p = jnp.concatenate([dst, N + pad_ids % TRASH]).reshape(ROWS, LANE)
  b1p = jnp.tile(b1, 8)
  x3 = x.reshape(PR, 8, D_IN)

  grid = ((PR + BLKR - 1) // BLKR,)

  h1p = pl.pallas_call(
      _ka_body,
      grid=grid,
      in_specs=[
          pl.BlockSpec((BLKR, 8, D_IN), lambda i: (i, 0, 0)),
          pl.BlockSpec((D_IN, D_HID), lambda i: (0, 0)),
      ],
      out_specs=pl.BlockSpec((BLKR, 128), lambda i: (i, 0)),
      out_shape=jax.ShapeDtypeStruct((PR, 128), jnp.float32),
  )(x3, W1)

  degr = _deg_call(dst_p).reshape(NC, PRA, 128)

  g1p, disp = pl.pallas_call(
      _kb_body,
      grid=grid,
      in_specs=[
          pl.BlockSpec((BLKR, 128), lambda i: (i, 0)),
          pl.BlockSpec((NC, BLKR, 128), lambda i: (0, i, 0)),
      ],
      out_specs=[
          pl.BlockSpec((BLKR, 128), lambda i: (i, 0)),
          pl.BlockSpec((BLKR, 128), lambda i: (i, 0)),
      ],
      out_shape=[
          jax.ShapeDtypeStruct((PR, 128), jnp.float32),
          jax.ShapeDtypeStruct((PR, 128), jnp.float32),
      ],
  )(h1p, degr)

  acc1 = _agg_call(g1p.reshape(N, D_HID), src_p, dst_p).reshape(NC, PRA, 128)

  g1rp = pl.pallas_call(
      _k4_body,
      grid=grid,
      in_specs=[
          pl.BlockSpec((NC, BLKR, 128), lambda i: (0, i, 0)),
          pl.BlockSpec((BLKR, 128), lambda i: (i, 0)),
          pl.BlockSpec((BLKR, 128), lambda i: (i, 0)),
          pl.BlockSpec((128,), lambda i: (0,)),
      ],
      out_specs=pl.BlockSpec((BLKR, 128), lambda i: (i, 0)),
      out_shape=jax.ShapeDtypeStruct((PR, 128), jnp.float32),
  )(acc1, g1p, disp, b1p)

  acc2 = _agg_call(g1rp.reshape(N, D_HID), src_p, dst_p).reshape(NC, PRA, 128)

  out = pl.pallas_call(
      _k6_body,
      grid=grid,
      in_specs=[
          pl.BlockSpec((NC, BLKR, 128), lambda i: (0, i, 0)),
          pl.BlockSpec((BLKR, 128), lambda i: (i, 0)),
          pl.BlockSpec((BLKR, 128), lambda i: (i, 0)),
          pl.BlockSpec((D_HID, D_OUT), lambda i: (0, 0)),
          pl.BlockSpec((D_OUT,), lambda i: (0,)),
      ],
      out_specs=pl.BlockSpec((8 * BLKR, D_OUT), lambda i: (i, 0)),
      out_shape=jax.ShapeDtypeStruct((N, D_OUT), jnp.float32),
  )(acc2, g1rp, disp, W2, b2)

  return out


# speedup vs baseline: 1.0619x; 1.0619x over previous
import jax
import jax.numpy as jnp
from jax import lax
from jax.experimental import pallas as pl
from jax.experimental.pallas import tpu as pltpu
import jax.experimental.pallas.tpu_sc as plsc

N = 100000
E = 3200000
D_IN = 128
D_HID = 16
D_OUT = 40

NC = 2
NS = 16
LANE = 128
ROWS = E // LANE
RW_LO = ROWS // (NC * NS)
NEXTRA = ROWS - RW_LO * NC * NS
NBLK = RW_LO // 40
TAIL = 22
NPAD = 102400
NSLICE = NPAD // NS

CH = 8
IDXB = 40
ZROWS = 100
RV = 400

PR = N * D_HID // 128
PRA = NPAD * D_HID // 128
BLKR = 256


def _deg_body(dst_hbm, out_hbm, accum, zbuf, ones, dstbuf, deg_v, rep_v, sem):
  c = lax.axis_index("c")
  s = lax.axis_index("s")

  @pl.loop(0, NSLICE // 16)
  def _zfill(i):
    zbuf[pl.ds(i * 16, 16)] = jnp.zeros((16,), jnp.float32)

  @pl.loop(0, LANE // 16)
  def _ofill(i):
    ones[pl.ds(i * 16, 16)] = jnp.ones((16,), jnp.float32)

  pltpu.sync_copy(zbuf, accum.at[pl.ds(s * NSLICE, NSLICE)])
  plsc.subcore_barrier()

  w = c * NS + s
  base = w * RW_LO + jnp.minimum(w, NEXTRA)
  rows_w = RW_LO + jnp.where(w < NEXTRA, 1, 0)

  def _scatter_ones(off, cnt):
    descs = []
    for j in range(cnt):
      descs.append(pltpu.async_copy(ones, accum.at[dstbuf.at[off + j]], sem,
                                    add=True))
    for d in descs:
      d.wait()

  @pl.loop(0, NBLK)
  def _blk(blk):
    pltpu.sync_copy(dst_hbm.at[pl.ds(base + blk * 40, 40)], dstbuf)

    @pl.loop(0, 2)
    def _chunk(sub):
      _scatter_ones(sub * 20, 20)

  pltpu.sync_copy(dst_hbm.at[pl.ds(base + rows_w - TAIL, TAIL)],
                  dstbuf.at[pl.ds(0, TAIL)])

  @pl.when(w >= NEXTRA)
  def _fix():
    for k in range(LANE // 16):
      dstbuf[0, pl.ds(k * 16, 16)] = (
          N + k * 16 + lax.iota(jnp.int32, 16))

  _scatter_ones(0, TAIL)
  plsc.subcore_barrier()

  @pl.loop(0, NSLICE // RV)
  def _rep(it):
    pltpu.sync_copy(accum.at[pl.ds(s * NSLICE + it * RV, RV)], deg_v)

    @pl.loop(0, RV // 16)
    def _grp(g):
      v = deg_v[pl.ds(g * 16, 16)]
      for j in range(16):
        rep_v[pl.ds((g * 16 + j) * 16, 16)] = v.at[
            jnp.full((16,), j, jnp.int32)].get(mode="promise_in_bounds")

    pltpu.sync_copy(
        rep_v,
        out_hbm.at[pl.ds((c * NPAD + s * NSLICE + it * RV) * D_HID,
                         RV * D_HID)])


def _agg_body(table_hbm, src_hbm, dst_hbm, out_hbm, accum, zbuf, srcbuf,
              dstbuf, rowbuf, gsem, ssem):
  c = lax.axis_index("c")
  s = lax.axis_index("s")

  @pl.loop(0, ZROWS)
  def _zfill(i):
    zbuf[i, :] = jnp.zeros((D_HID,), jnp.float32)

  @pl.loop(0, NSLICE // ZROWS)
  def _zinit(j):
    pltpu.sync_copy(zbuf, accum.at[pl.ds(s * NSLICE + j * ZROWS, ZROWS), :])

  plsc.subcore_barrier()

  w = c * NS + s
  base = w * RW_LO + jnp.minimum(w, NEXTRA)
  rows_w = RW_LO + jnp.where(w < NEXTRA, 1, 0)

  def _gs(off, cnt):
    gd = []
    for j in range(cnt):
      gd.append(pltpu.async_copy(table_hbm.at[srcbuf.at[off + j]],
                                 rowbuf.at[j], gsem))
    sd = []
    for j in range(cnt):
      gd[j].wait()
      sd.append(pltpu.async_copy(rowbuf.at[j],
                                 accum.at[dstbuf.at[off + j]], ssem,
                                 add=True))
    for d in sd:
      d.wait()

  @pl.loop(0, NBLK)
  def _blk(blk):
    r0 = base + blk * 40
    pltpu.sync_copy(src_hbm.at[pl.ds(r0, 40)], srcbuf)
    pltpu.sync_copy(dst_hbm.at[pl.ds(r0, 40)], dstbuf)

    @pl.loop(0, 40 // CH)
    def _chunk(sub):
      _gs(sub * CH, CH)

  t0 = base + rows_w - TAIL
  pltpu.sync_copy(src_hbm.at[pl.ds(t0, TAIL)], srcbuf.at[pl.ds(0, TAIL)])
  pltpu.sync_copy(dst_hbm.at[pl.ds(t0, TAIL)], dstbuf.at[pl.ds(0, TAIL)])

  @pl.when(w >= NEXTRA)
  def _fix():
    for k in range(LANE // 16):
      dstbuf[0, pl.ds(k * 16, 16)] = (
          N + k * 16 + lax.iota(jnp.int32, 16))

  _gs(0, 8)
  _gs(8, 8)
  _gs(16, TAIL - 16)
  plsc.subcore_barrier()
  pltpu.sync_copy(accum.at[pl.ds(s * NSLICE, NSLICE), :],
                  out_hbm.at[pl.ds(c * NPAD + s * NSLICE, NSLICE), :])


def _ka_body(x_ref, w1_ref, h_ref):
  for j in range(8):
    h_ref[:, pl.ds(D_HID * j, D_HID)] = jnp.dot(
        x_ref[:, j, :], w1_ref[...], preferred_element_type=jnp.float32)


def _kb_body(h_ref, degr_ref, g1_ref, dis_ref):
  degsum = degr_ref[0] + degr_ref[1] + 1.0
  dis = lax.rsqrt(degsum)
  g1_ref[...] = h_ref[...] * dis
  dis_ref[...] = dis


def _k4_body(a_ref, g1_ref, dis_ref, b1p_ref, o_ref):
  dis = dis_ref[...]
  t = dis * (a_ref[0] + a_ref[1] + g1_ref[...]) + b1p_ref[...]
  o_ref[...] = dis * jnp.maximum(t, 0.0)


def _k6_body(a_ref, g1r_ref, dis_ref, w2_ref, b2_ref, o_ref):
  m = dis_ref[...] * (a_ref[0] + a_ref[1] + g1r_ref[...])
  cols = [
      jnp.dot(m[:, D_HID * j:D_HID * (j + 1)], w2_ref[...],
              preferred_element_type=jnp.float32)
      for j in range(8)
  ]
  stacked = jnp.stack(cols, axis=1)
  o_ref[...] = stacked.reshape(8 * BLKR, D_OUT) + b2_ref[...]


def _sc_mesh():
  return plsc.VectorSubcoreMesh(core_axis_name="c", subcore_axis_name="s",
                                num_cores=NC, num_subcores=NS)


_SC_PARAMS = pltpu.CompilerParams(use_tc_tiling_on_sc=False)


def _deg_call(dst_p):
  fn = pl.kernel(
      _deg_body,
      out_type=jax.ShapeDtypeStruct((NC * NPAD * D_HID,), jnp.float32),
      mesh=_sc_mesh(),
      compiler_params=_SC_PARAMS,
      scratch_types=[
          pltpu.VMEM_SHARED((NPAD,), jnp.float32),
          pltpu.VMEM((NSLICE,), jnp.float32),
          pltpu.VMEM((LANE,), jnp.float32),
          pltpu.VMEM((IDXB, LANE), jnp.int32),
          pltpu.VMEM((RV,), jnp.float32),
          pltpu.VMEM((RV * D_HID,), jnp.float32),
          pltpu.SemaphoreType.DMA,
      ],
  )
  return fn(dst_p)


def _agg_call(table, src_p, dst_p):
  fn = pl.kernel(
      _agg_body,
      out_type=jax.ShapeDtypeStruct((NC * NPAD, D_HID), jnp.float32),
      mesh=_sc_mesh(),
      compiler_params=_SC_PARAMS,
      scratch_types=[
          pltpu.VMEM_SHARED((NPAD, D_HID), jnp.float32),
          pltpu.VMEM((ZROWS, D_HID), jnp.float32),
          pltpu.VMEM((IDXB, LANE), jnp.int32),
          pltpu.VMEM((IDXB, LANE), jnp.int32),
          pltpu.VMEM((CH, LANE, D_HID), jnp.float32),
          pltpu.SemaphoreType.DMA,
          pltpu.SemaphoreType.DMA,
      ],
  )
  return fn(table, src_p, dst_p)


def kernel(x, edge_index, W1, b1, W2, b2):
  src = edge_index[0].astype(jnp.int32)
  dst = edge_index[1].astype(jnp.int32)
  src_p = src.reshape(ROWS, LANE)
  dst_p = dst.reshape(ROWS, LANE)
  b1p = jnp.tile(b1, 8)
  x3 = x.reshape(PR, 8, D_IN)

  grid = ((PR + BLKR - 1) // BLKR,)

  h1p = pl.pallas_call(
      _ka_body,
      grid=grid,
      in_specs=[
          pl.BlockSpec((BLKR, 8, D_IN), lambda i: (i, 0, 0)),
          pl.BlockSpec((D_IN, D_HID), lambda i: (0, 0)),
      ],
      out_specs=pl.BlockSpec((BLKR, 128), lambda i: (i, 0)),
      out_shape=jax.ShapeDtypeStruct((PR, 128), jnp.float32),
  )(x3, W1)

  degr = _deg_call(dst_p).reshape(NC, PRA, 128)

  g1p, disp = pl.pallas_call(
      _kb_body,
      grid=grid,
      in_specs=[
          pl.BlockSpec((BLKR, 128), lambda i: (i, 0)),
          pl.BlockSpec((NC, BLKR, 128), lambda i: (0, i, 0)),
      ],
      out_specs=[
          pl.BlockSpec((BLKR, 128), lambda i: (i, 0)),
          pl.BlockSpec((BLKR, 128), lambda i: (i, 0)),
      ],
      out_shape=[
          jax.ShapeDtypeStruct((PR, 128), jnp.float32),
          jax.ShapeDtypeStruct((PR, 128), jnp.float32),
      ],
  )(h1p, degr)

  acc1 = _agg_call(g1p.reshape(N, D_HID), src_p, dst_p).reshape(NC, PRA, 128)

  g1rp = pl.pallas_call(
      _k4_body,
      grid=grid,
      in_specs=[
          pl.BlockSpec((NC, BLKR, 128), lambda i: (0, i, 0)),
          pl.BlockSpec((BLKR, 128), lambda i: (i, 0)),
          pl.BlockSpec((BLKR, 128), lambda i: (i, 0)),
          pl.BlockSpec((128,), lambda i: (0,)),
      ],
      out_specs=pl.BlockSpec((BLKR, 128), lambda i: (i, 0)),
      out_shape=jax.ShapeDtypeStruct((PR, 128), jnp.float32),
  )(acc1, g1p, disp, b1p)

  acc2 = _agg_call(g1rp.reshape(N, D_HID), src_p, dst_p).reshape(NC, PRA, 128)

  out = pl.pallas_call(
      _k6_body,
      grid=grid,
      in_specs=[
          pl.BlockSpec((NC, BLKR, 128), lambda i: (0, i, 0)),
          pl.BlockSpec((BLKR, 128), lambda i: (i, 0)),
          pl.BlockSpec((BLKR, 128), lambda i: (i, 0)),
          pl.BlockSpec((D_HID, D_OUT), lambda i: (0, 0)),
          pl.BlockSpec((D_OUT,), lambda i: (0,)),
      ],
      out_specs=pl.BlockSpec((8 * BLKR, D_OUT), lambda i: (i, 0)),
      out_shape=jax.ShapeDtypeStruct((N, D_OUT), jnp.float32),
  )(acc2, g1rp, disp, W2, b2)

  return out
